# Initial kernel scaffold; baseline (speedup 1.0000x reference)
import jax
import jax.numpy as jnp
from jax.experimental import pallas as pl


def kernel(x, batch):
    raise NotImplementedError("write your pallas kernel here")


# speedup vs baseline: 23.1444x; 23.1444x over previous
import functools

import jax
import jax.numpy as jnp
from jax import lax
from jax.experimental import pallas as pl
from jax.experimental.pallas import tpu as pltpu
from jax.experimental.pallas import tpu_sc as plsc

_N = 6_400_000
_S = 4096
_L = 16
_NC = 2
_NS = 16
_NW = _NC * _NS
_RPW = _N // _NW
_CHUNK = 4000
_NCHUNK = _RPW // _CHUNK
_CONST = 1.0

_mesh = plsc.VectorSubcoreMesh(core_axis_name="c", subcore_axis_name="s")


@functools.partial(
    pl.kernel,
    out_type=jax.ShapeDtypeStruct((_NW, _S * _L), jnp.float32),
    mesh=_mesh,
    scratch_types=[
        pltpu.VMEM((_CHUNK,), jnp.float32),
        pltpu.VMEM((_CHUNK,), jnp.int32),
        pltpu.VMEM((_S * _L,), jnp.float32),
    ],
    compiler_params=pltpu.CompilerParams(needs_layout_passes=False),
)
def _seg_partial(x_hbm, b_hbm, part_hbm, xb, bb, acc):
    c = lax.axis_index("c")
    s = lax.axis_index("s")
    wid = c * _NS + s
    base = wid * _RPW

    zeros = jnp.zeros((_L,), jnp.float32)

    def zbody(i, carry):
        acc[pl.ds(i * _L, _L)] = zeros
        return carry

    lax.fori_loop(0, _S, zbody, 0)

    lane = lax.iota(jnp.int32, _L)

    def chunk_body(j, carry):
        off = base + j * _CHUNK
        pltpu.sync_copy(x_hbm.at[pl.ds(off, _CHUNK)], xb)
        pltpu.sync_copy(b_hbm.at[pl.ds(off, _CHUNK)], bb)

        def vbody(i, inner):
            v = xb[pl.ds(i * _L, _L)]
            ids = bb[pl.ds(i * _L, _L)]
            plsc.addupdate_scatter(acc, [ids * _L + lane], v)
            return inner

        lax.fori_loop(0, _CHUNK // _L, vbody, 0)
        return carry

    lax.fori_loop(0, _NCHUNK, chunk_body, 0)
    pltpu.sync_copy(acc, part_hbm.at[wid])


def _fold_body(p_ref, o_ref):
    o_ref[...] = jnp.sum(p_ref[...], axis=(0, 2)) * _CONST


def _fold(part):
    blk = _S // 8
    return pl.pallas_call(
        _fold_body,
        grid=(8,),
        in_specs=[pl.BlockSpec((_NW, blk, _L), lambda g: (0, g, 0))],
        out_specs=pl.BlockSpec((blk,), lambda g: (g,)),
        out_shape=jax.ShapeDtypeStruct((_S,), jnp.float32),
    )(part)


def kernel(x, batch):
    xf = x.reshape(-1)
    part = _seg_partial(xf, batch)
    return _fold(part.reshape(_NW, _S, _L))


# speedup vs baseline: 33.8929x; 1.4644x over previous
import functools

import jax
import jax.numpy as jnp
from jax import lax
from jax.experimental import pallas as pl
from jax.experimental.pallas import tpu as pltpu
from jax.experimental.pallas import tpu_sc as plsc

_N = 6_400_000
_S = 4096
_L = 16
_NC = 2
_NS = 16
_NW = _NC * _NS
_RPW = _N // _NW
_CHUNK = 10000
_NCHUNK = _RPW // _CHUNK
_NBUF = 2
_UNROLL = 5
_CONST = 1.0

_mesh = plsc.VectorSubcoreMesh(core_axis_name="c", subcore_axis_name="s")


@functools.partial(
    pl.kernel,
    out_type=jax.ShapeDtypeStruct((_NW, _S * _L), jnp.float32),
    mesh=_mesh,
    scratch_types=[
        pltpu.VMEM((_CHUNK,), jnp.float32),
        pltpu.VMEM((_CHUNK,), jnp.float32),
        pltpu.VMEM((_CHUNK,), jnp.int32),
        pltpu.VMEM((_CHUNK,), jnp.int32),
        pltpu.VMEM((_S * _L,), jnp.float32),
        pltpu.SemaphoreType.DMA,
        pltpu.SemaphoreType.DMA,
    ],
    compiler_params=pltpu.CompilerParams(needs_layout_passes=False),
)
def _seg_partial(x_hbm, b_hbm, part_hbm, xb0, xb1, bb0, bb1, acc, sem0, sem1):
    c = lax.axis_index("c")
    s = lax.axis_index("s")
    wid = c * _NS + s
    base = wid * _RPW
    xbufs = [xb0, xb1]
    bbufs = [bb0, bb1]
    sems = [sem0, sem1]

    zeros = jnp.zeros((_L,), jnp.float32)

    def zbody(i, carry):
        for u in range(16):
            acc[pl.ds((i * 16 + u) * _L, _L)] = zeros
        return carry

    lax.fori_loop(0, _S // 16, zbody, 0)

    lane = lax.iota(jnp.int32, _L)

    def start(j, b):
        off = base + j * _CHUNK
        pltpu.async_copy(x_hbm.at[pl.ds(off, _CHUNK)], xbufs[b], sems[b])
        pltpu.async_copy(b_hbm.at[pl.ds(off, _CHUNK)], bbufs[b], sems[b])

    def wait(j, b):
        off = base + j * _CHUNK
        pltpu.make_async_copy(x_hbm.at[pl.ds(off, _CHUNK)], xbufs[b], sems[b]).wait()
        pltpu.make_async_copy(b_hbm.at[pl.ds(off, _CHUNK)], bbufs[b], sems[b]).wait()

    def compute(b):
        xrow = xbufs[b]
        brow = bbufs[b]

        def vbody(i, inner):
            for u in range(_UNROLL):
                o = (i * _UNROLL + u) * _L
                v = xrow[pl.ds(o, _L)]
                ids = brow[pl.ds(o, _L)]
                plsc.addupdate_scatter(acc, [ids * _L + lane], v)
            return inner

        lax.fori_loop(0, _CHUNK // (_L * _UNROLL), vbody, 0)

    for b in range(_NBUF):
        start(b, b)

    def outer(t, carry):
        j0 = t * _NBUF
        for b in range(_NBUF):
            j = j0 + b
            wait(j, b)
            compute(b)

            @pl.when(j + _NBUF < _NCHUNK)
            def _():
                start(j + _NBUF, b)

        return carry

    lax.fori_loop(0, _NCHUNK // _NBUF, outer, 0)
    pltpu.sync_copy(acc, part_hbm.at[wid])


def _fold_body(p_ref, o_ref):
    o_ref[...] = jnp.sum(p_ref[...], axis=(0, 2)) * _CONST


def _fold(part):
    blk = _S // 8
    return pl.pallas_call(
        _fold_body,
        grid=(8,),
        in_specs=[pl.BlockSpec((_NW, blk, _L), lambda g: (0, g, 0))],
        out_specs=pl.BlockSpec((blk,), lambda g: (g,)),
        out_shape=jax.ShapeDtypeStruct((_S,), jnp.float32),
    )(part)


def kernel(x, batch):
    xf = x.reshape(-1)
    part = _seg_partial(xf, batch)
    return _fold(part.reshape(_NW, _S, _L))


# speedup vs baseline: 53.5969x; 1.5814x over previous
import functools

import jax
import jax.numpy as jnp
from jax import lax
from jax.experimental import pallas as pl
from jax.experimental.pallas import tpu as pltpu
from jax.experimental.pallas import tpu_sc as plsc

_N = 6_400_000
_S = 4096
_L = 16
_NC = 2
_NS = 16
_NW = _NC * _NS
_RPW = _N // _NW
_CHUNK = 10000
_NCHUNK = _RPW // _CHUNK
_NBUF = 2
_UNROLL = 5
_CONST = 1.0

_mesh = plsc.VectorSubcoreMesh(core_axis_name="c", subcore_axis_name="s")


@functools.partial(
    pl.kernel,
    out_type=jax.ShapeDtypeStruct((_NW, _S * _L), jnp.float32),
    mesh=_mesh,
    scratch_types=[
        pltpu.VMEM((_CHUNK,), jnp.float32),
        pltpu.VMEM((_CHUNK,), jnp.float32),
        pltpu.VMEM((_CHUNK,), jnp.int32),
        pltpu.VMEM((_CHUNK,), jnp.int32),
        pltpu.VMEM((_S * _L,), jnp.float32),
        pltpu.SemaphoreType.DMA,
        pltpu.SemaphoreType.DMA,
    ],
    compiler_params=pltpu.CompilerParams(needs_layout_passes=False),
)
def _seg_partial(x_hbm, b_hbm, part_hbm, xb0, xb1, bb0, bb1, acc, sem0, sem1):
    c = lax.axis_index("c")
    s = lax.axis_index("s")
    wid = c * _NS + s
    base = wid * _RPW
    xbufs = [xb0, xb1]
    bbufs = [bb0, bb1]
    sems = [sem0, sem1]

    zeros = jnp.zeros((_L,), jnp.float32)

    def zbody(i, carry):
        for u in range(16):
            acc[pl.ds((i * 16 + u) * _L, _L)] = zeros
        return carry

    lax.fori_loop(0, _S // 16, zbody, 0)

    lane = lax.iota(jnp.int32, _L)

    def start(j, b):
        off = base + j * _CHUNK
        pltpu.async_copy(x_hbm.at[pl.ds(off, _CHUNK)], xbufs[b], sems[b])
        pltpu.async_copy(b_hbm.at[pl.ds(off, _CHUNK)], bbufs[b], sems[b])

    def wait(j, b):
        off = base + j * _CHUNK
        pltpu.make_async_copy(x_hbm.at[pl.ds(off, _CHUNK)], xbufs[b], sems[b]).wait()
        pltpu.make_async_copy(b_hbm.at[pl.ds(off, _CHUNK)], bbufs[b], sems[b]).wait()

    def compute(b):
        xrow = xbufs[b]
        brow = bbufs[b]

        def vbody(i, inner):
            for u in range(_UNROLL):
                o = (i * _UNROLL + u) * _L
                v = xrow[pl.ds(o, _L)]
                ids = brow[pl.ds(o, _L)]
                plsc.addupdate_scatter(acc, [ids * _L + lane], v)
            return inner

        lax.fori_loop(0, _CHUNK // (_L * _UNROLL), vbody, 0)

    for b in range(_NBUF):
        start(b, b)

    def outer(t, carry):
        j0 = t * _NBUF
        for b in range(_NBUF):
            j = j0 + b
            wait(j, b)
            compute(b)

            @pl.when(j + _NBUF < _NCHUNK)
            def _():
                start(j + _NBUF, b)

        return carry

    lax.fori_loop(0, _NCHUNK // _NBUF, outer, 0)
    pltpu.sync_copy(acc, part_hbm.at[wid])


def _fold_body(p_ref, o_ref):
    blk = o_ref.shape[0]
    p = p_ref[...].reshape(_NW, blk, _L)
    o_ref[...] = jnp.sum(p, axis=(0, 2)) * _CONST


def _fold(part):
    blk = _S // 8
    return pl.pallas_call(
        _fold_body,
        grid=(8,),
        in_specs=[pl.BlockSpec((_NW, blk * _L), lambda g: (0, g))],
        out_specs=pl.BlockSpec((blk,), lambda g: (g,)),
        out_shape=jax.ShapeDtypeStruct((_S,), jnp.float32),
    )(part)


def kernel(x, batch):
    xf = x.reshape(-1)
    part = _seg_partial(xf, batch)
    return _fold(part)


# speedup vs baseline: 54.9565x; 1.0254x over previous
import functools

import jax
import jax.numpy as jnp
from jax import lax
from jax.experimental import pallas as pl
from jax.experimental.pallas import tpu as pltpu
from jax.experimental.pallas import tpu_sc as plsc

_N = 6_400_000
_S = 4096
_L = 16
_NC = 2
_NS = 16
_NW = _NC * _NS
_RPW = _N // _NW
_CHUNK = 10000
_NCHUNK = _RPW // _CHUNK
_NBUF = 2
_UNROLL = 25
_CONST = 1.0

_mesh = plsc.VectorSubcoreMesh(core_axis_name="c", subcore_axis_name="s")


@functools.partial(
    pl.kernel,
    out_type=jax.ShapeDtypeStruct((_NW, _S * _L), jnp.float32),
    mesh=_mesh,
    scratch_types=[
        pltpu.VMEM((_CHUNK,), jnp.float32),
        pltpu.VMEM((_CHUNK,), jnp.float32),
        pltpu.VMEM((_CHUNK,), jnp.int32),
        pltpu.VMEM((_CHUNK,), jnp.int32),
        pltpu.VMEM((_S * _L,), jnp.float32),
        pltpu.SemaphoreType.DMA,
        pltpu.SemaphoreType.DMA,
    ],
    compiler_params=pltpu.CompilerParams(needs_layout_passes=False),
)
def _seg_partial(x_hbm, b_hbm, part_hbm, xb0, xb1, bb0, bb1, acc, sem0, sem1):
    c = lax.axis_index("c")
    s = lax.axis_index("s")
    wid = c * _NS + s
    base = wid * _RPW
    xbufs = [xb0, xb1]
    bbufs = [bb0, bb1]
    sems = [sem0, sem1]

    zeros = jnp.zeros((_L,), jnp.float32)

    def zbody(i, carry):
        for u in range(16):
            acc[pl.ds((i * 16 + u) * _L, _L)] = zeros
        return carry

    lax.fori_loop(0, _S // 16, zbody, 0)

    lane = lax.iota(jnp.int32, _L)

    def start(j, b):
        off = base + j * _CHUNK
        pltpu.async_copy(x_hbm.at[pl.ds(off, _CHUNK)], xbufs[b], sems[b])
        pltpu.async_copy(b_hbm.at[pl.ds(off, _CHUNK)], bbufs[b], sems[b])

    def wait(j, b):
        off = base + j * _CHUNK
        pltpu.make_async_copy(x_hbm.at[pl.ds(off, _CHUNK)], xbufs[b], sems[b]).wait()
        pltpu.make_async_copy(b_hbm.at[pl.ds(off, _CHUNK)], bbufs[b], sems[b]).wait()

    def compute(b):
        xrow = xbufs[b]
        brow = bbufs[b]

        def vbody(i, inner):
            for u in range(_UNROLL):
                o = (i * _UNROLL + u) * _L
                v = xrow[pl.ds(o, _L)]
                ids = brow[pl.ds(o, _L)]
                plsc.addupdate_scatter(acc, [ids * _L + lane], v)
            return inner

        lax.fori_loop(0, _CHUNK // (_L * _UNROLL), vbody, 0)

    for b in range(_NBUF):
        start(b, b)

    def outer(t, carry):
        j0 = t * _NBUF
        for b in range(_NBUF):
            j = j0 + b
            wait(j, b)
            compute(b)

            @pl.when(j + _NBUF < _NCHUNK)
            def _():
                start(j + _NBUF, b)

        return carry

    lax.fori_loop(0, _NCHUNK // _NBUF, outer, 0)
    pltpu.sync_copy(acc, part_hbm.at[wid])


def _fold_body(p_ref, o_ref):
    blk = o_ref.shape[0]
    p = p_ref[...].reshape(_NW, blk, _L)
    o_ref[...] = jnp.sum(p, axis=(0, 2)) * _CONST


def _fold(part):
    blk = _S // 8
    return pl.pallas_call(
        _fold_body,
        grid=(8,),
        in_specs=[pl.BlockSpec((_NW, blk * _L), lambda g: (0, g))],
        out_specs=pl.BlockSpec((blk,), lambda g: (g,)),
        out_shape=jax.ShapeDtypeStruct((_S,), jnp.float32),
    )(part)


def kernel(x, batch):
    xf = x.reshape(-1)
    part = _seg_partial(xf, batch)
    return _fold(part)
